# baseline (speedup 1.0000x reference)
import jax
import jax.numpy as jnp
from jax import lax
from jax.experimental import pallas as pl
from jax.experimental.pallas import tpu as pltpu
from jax.experimental.pallas import tpu_sc as plsc

B = 16384
D = 64
NC = 2
NS = 16
NW = NC * NS
BPW = B // NW
CH = 128
NCH = BPW // CH
NBIAS = 8192


def _project_kernel(cat_ref, w_ref, out_ref):
    out_ref[...] = jnp.dot(
        cat_ref[...], w_ref[...].T, preferred_element_type=jnp.float32
    )


def _project(category_x, cat2item_w):
    n_cat = category_x.shape[0]
    return pl.pallas_call(
        _project_kernel,
        out_shape=jax.ShapeDtypeStruct((n_cat, D), jnp.float32),
    )(category_x, cat2item_w)


def _sc_body(uidx_hbm, iidx_hbm, cidx_hbm, uw_hbm, ub_hbm, ix_hbm, pj_hbm,
             out_hbm, uidx_v, iidx_v, cidx_v, uli_v, ili_v, cli_v, bli_v,
             w_v, x_v, p_v, b_v, out_v, sem):
    wid = lax.axis_index("s") * NC + lax.axis_index("c")
    pltpu.sync_copy(uidx_hbm.at[wid], uidx_v)
    pltpu.sync_copy(iidx_hbm.at[wid], iidx_v)
    pltpu.sync_copy(cidx_hbm.at[wid], cidx_v)
    lane = lax.iota(jnp.int32, 16)

    for j in range(NCH):
        for o in range(CH // 16):
            s = pl.ds(o * 16, 16)
            fl = pl.ds(j * CH + o * 16, 16)
            uvec = uidx_v[fl]
            uli_v[j, s] = lax.shift_right_logical(uvec, 1)
            bli_v[j, s] = lax.shift_right_logical(uvec, 7)
            ili_v[j, s] = lax.shift_right_logical(iidx_v[fl], 1)
            cli_v[j, s] = lax.shift_right_logical(cidx_v[fl], 1)

    def chunk(c, _):
        base = c * CH

        copies = [
            pltpu.async_copy(uw_hbm.at[uli_v.at[c]], w_v, sem),
            pltpu.async_copy(ix_hbm.at[ili_v.at[c]], x_v, sem),
            pltpu.async_copy(pj_hbm.at[cli_v.at[c]], p_v, sem),
            pltpu.async_copy(ub_hbm.at[bli_v.at[c]], b_v, sem),
        ]
        for cp in copies:
            cp.wait()

        def group(g, _):
            res = jnp.zeros((16,), jnp.float32)
            gb = g * 16
            fl = pl.ds(base + gb, 16)
            uvec = uidx_v[fl]
            ivec = iidx_v[fl]
            cvec = cidx_v[fl]
            for r in range(16):
                i = gb + r
                wo = (uvec[r] & 1) * D
                xo = (ivec[r] & 1) * D
                po = (cvec[r] & 1) * D
                acc = w_v[i, pl.ds(wo, 16)] * (
                    x_v[i, pl.ds(xo, 16)] + p_v[i, pl.ds(po, 16)]
                )
                for k in range(1, D // 16):
                    acc = acc + w_v[i, pl.ds(wo + k * 16, 16)] * (
                        x_v[i, pl.ds(xo + k * 16, 16)]
                        + p_v[i, pl.ds(po + k * 16, 16)]
                    )
                res = jnp.where(lane == r, jnp.sum(acc), res)
            bias = plsc.load_gather(b_v, [gb + lane, uvec & 127])
            out_v[pl.ds(base + gb, 16)] = res + bias
            return 0

        lax.fori_loop(0, CH // 16, group, 0)
        return 0

    lax.fori_loop(0, NCH, chunk, 0)

    pltpu.sync_copy(out_v, out_hbm.at[pl.ds(wid * BPW, BPW)])


@jax.jit
def _sc_gather_dot(uidx2, iidx2, cidx2, uw2, ub128, ix2, pj2):
    mesh = plsc.VectorSubcoreMesh(core_axis_name="c", subcore_axis_name="s")
    return pl.kernel(
        _sc_body,
        out_type=jax.ShapeDtypeStruct((B,), jnp.float32),
        mesh=mesh,
        compiler_params=pltpu.CompilerParams(needs_layout_passes=False),
        scratch_types=[
            pltpu.VMEM((BPW,), jnp.int32),
            pltpu.VMEM((BPW,), jnp.int32),
            pltpu.VMEM((BPW,), jnp.int32),
            pltpu.VMEM((NCH, CH), jnp.int32),
            pltpu.VMEM((NCH, CH), jnp.int32),
            pltpu.VMEM((NCH, CH), jnp.int32),
            pltpu.VMEM((NCH, CH), jnp.int32),
            pltpu.VMEM((CH, 128), jnp.float32),
            pltpu.VMEM((CH, 128), jnp.float32),
            pltpu.VMEM((CH, 128), jnp.float32),
            pltpu.VMEM((CH, 128), jnp.float32),
            pltpu.VMEM((BPW,), jnp.float32),
            pltpu.SemaphoreType.DMA,
        ],
    )(uidx2, iidx2, cidx2, uw2, ub128, ix2, pj2)


def kernel(user_idx, item_idx, category_idx, user_w, user_b, item_x,
           category_x, cat2item_w):
    proj = _project(category_x, cat2item_w)
    uidx2 = user_idx.astype(jnp.int32).reshape(NW, BPW)
    iidx2 = item_idx.astype(jnp.int32).reshape(NW, BPW)
    cidx2 = category_idx.astype(jnp.int32).reshape(NW, BPW)
    uw2 = user_w.reshape(user_w.shape[0] // 2, 2 * D)
    ix2 = item_x.reshape(item_x.shape[0] // 2, 2 * D)
    pj2 = proj.reshape(proj.shape[0] // 2, 2 * D)
    nb = user_b.shape[0]
    ub128 = jnp.pad(user_b.reshape(nb), (0, NBIAS * 128 - nb)).reshape(NBIAS, 128)
    return _sc_gather_dot(uidx2, iidx2, cidx2, uw2, ub128, ix2, pj2)
